# baseline (speedup 1.0000x reference)
import math

import jax
import jax.numpy as jnp
from jax import lax
from jax.experimental import pallas as pl
from jax.experimental.pallas import tpu as pltpu
from jax.experimental.pallas import tpu_sc as plsc

_VOCAB = 1000
_D = 512
_B = 64
_L = 512
_SCALE = math.sqrt(float(_D))
_NEG_INF = float("-inf")

_NC = 2
_NS = 16
_NW = _NC * _NS
_CH = 32
_NP = _L // _CH
_LP = _L - 1
_NCK = 2 * _NP
_NT = _NCK // 4


def _mask16(iv):
    return jnp.where(iv == 0, jnp.float32(_NEG_INF), jnp.float32(0.0))


def _add_pos(rows_v, pos_v):
    def row_body(r, carry):
        for d in range(_D // 16):
            sl = pl.ds(d * 16, 16)
            plsc.addupdate(rows_v.at[r, sl], pos_v[r, sl])
        return carry
    lax.fori_loop(0, _CH, row_body, 0, unroll=False)


def _sc_body(x_hbm, y_hbm, emb_hbm, psrc_hbm, ptgt_hbm,
             xe_hbm, msk_hbm, ye_hbm, mtgt_hbm, emb_s,
             xidx_v, yidx_v,
             rows0, rows1, rows2, rows3, pos0, pos1,
             mskx_v, mska_v, mskb_v,
             sidx0, sidx1, sidx2, sidx3,
             sg0, sg1, sg2, sg3, ss0, ss1, ss2, ss3, sp0, sp1):
    wid = lax.axis_index("s") * _NC + lax.axis_index("c")
    sid = lax.axis_index("s")
    iota16 = lax.iota(jnp.int32, 16)
    rows = [rows0, rows1, rows2, rows3]
    pos = [pos0, pos1]
    sidx = [sidx0, sidx1, sidx2, sidx3]
    sg = [sg0, sg1, sg2, sg3]
    ss = [ss0, ss1, ss2, ss3]
    sp = [sp0, sp1]
    base1k = wid * (2 * _L)

    def wait_gather(s):
        pltpu.make_async_copy(psrc_hbm.at[pl.ds(0, _CH), :], rows[s],
                              sg[s]).wait()

    def wait_store(s):
        pltpu.make_async_copy(rows[s], xe_hbm.at[pl.ds(0, _CH), :],
                              ss[s]).wait()

    def wait_pos(par):
        pltpu.make_async_copy(psrc_hbm.at[pl.ds(0, _CH), :], pos[par],
                              sp[par]).wait()

    for u in (2 * sid, 2 * sid + 1):
        ebase = pl.multiple_of(
            jnp.minimum(u * _CH, _VOCAB - _CH).astype(jnp.int32), 8)
        pltpu.sync_copy(emb_hbm.at[pl.ds(ebase, _CH), :], rows0)

        def scale_row(r, carry):
            for d in range(_D // 16):
                sl = pl.ds(d * 16, 16)
                rows0[r, sl] = rows0[r, sl] * _SCALE
            return carry

        lax.fori_loop(0, _CH, scale_row, 0, unroll=False)
        pltpu.sync_copy(rows0, emb_s.at[pl.ds(ebase, _CH), :])
    pltpu.sync_copy(x_hbm.at[pl.ds(base1k, 2 * _L)], xidx_v)
    pltpu.sync_copy(y_hbm.at[pl.ds(base1k, 2 * _L)], yidx_v)
    plsc.subcore_barrier()

    def run_loop(idx_v, pos_hbm, p_eff_fn, store_fn, mask_fn):
        def idx_off(p_lin, bi):
            l0 = p_eff_fn(p_lin) * _CH
            return pl.multiple_of(bi * _L + l0, _CH), l0

        def issue_gather(p_lin, bi, s):
            o_i, _ = idx_off(p_lin, bi)
            pltpu.async_copy(emb_s.at[idx_v.at[pl.ds(o_i, _CH)]],
                             rows[s], sg[s])

        def issue_pos(p_lin, par):
            l0 = pl.multiple_of(p_eff_fn(p_lin) * _CH, _CH)
            pltpu.async_copy(pos_hbm.at[pl.ds(l0, _CH), :],
                             pos[par], sp[par])

        issue_pos(0, 0)
        issue_gather(0, 0, 0)
        issue_gather(0, 1, 1)

        def group(t, carry):
            for s in range(4):
                s_half = s // 2
                bi = s % 2
                p_lin = 2 * t + s_half
                o_i, l0 = idx_off(p_lin, bi)
                if s == 0:
                    wait_pos(0)
                if s == 2:
                    wait_pos(1)
                wait_gather(s)
                _add_pos(rows[s], pos[s_half])
                store_fn(p_lin, bi, s, o_i, l0)
                mask_fn(bi, o_i, l0)
                if s == 0:
                    issue_pos(2 * t + 1, 1)
                if s == 2:
                    @pl.when(t < _NT - 1)
                    def _():
                        issue_pos(2 * t + 2, 0)
                j = (s + 2) % 4
                if s in (0, 1):
                    @pl.when(t > 0)
                    def _():
                        wait_store(j)
                    issue_gather(2 * t + 1, bi, j)
                else:
                    wait_store(j)

                    @pl.when(t < _NT - 1)
                    def _():
                        issue_gather(2 * t + 2, bi, j)
            return carry

        lax.fori_loop(0, _NT, group, 0, unroll=False)
        wait_store(2)
        wait_store(3)

    def xe_p_eff(p_lin):
        return p_lin

    def xe_store(p_lin, bi, s, o_i, l0):
        dst = pl.multiple_of(base1k + o_i, _CH)
        pltpu.async_copy(rows[s], xe_hbm.at[pl.ds(dst, _CH), :], ss[s])

    def xe_mask(bi, o_i, l0):
        for j in range(_CH // 16):
            mskx_v[pl.ds(o_i + j * 16, 16)] = _mask16(
                xidx_v[pl.ds(o_i + j * 16, 16)])

    run_loop(xidx_v, psrc_hbm, xe_p_eff, xe_store, xe_mask)
    pltpu.sync_copy(mskx_v, msk_hbm.at[pl.ds(base1k, 2 * _L)])

    def ye_p_eff(p_lin):
        return jnp.where(
            p_lin == 0, _NP - 1,
            jnp.where(p_lin == 1, 1, jnp.where(p_lin == 2, 0, p_lin - 1)))

    def ye_store(p_lin, bi, s, o_i, l0):
        b = 2 * wid + bi
        for j in range(_CH // 16):
            posv = l0 + j * 16 + iota16
            sidx[s][j * 16:(j + 1) * 16] = jnp.where(
                posv == _L - 1, b, posv * _B + b)
        pltpu.async_copy(rows[s], ye_hbm.at[sidx[s]], ss[s])

    def ye_mask(bi, o_i, l0):
        mv = mska_v if bi == 0 else mskb_v
        for j in range(_CH // 16):
            mv[pl.ds(l0 + j * 16, 16)] = _mask16(
                yidx_v[pl.ds(o_i + j * 16, 16)])

    run_loop(yidx_v, ptgt_hbm, ye_p_eff, ye_store, ye_mask)
    pltpu.sync_copy(mska_v.at[pl.ds(0, _LP)], mtgt_hbm.at[2 * wid, 0, :])
    pltpu.sync_copy(mskb_v.at[pl.ds(0, _LP)],
                    mtgt_hbm.at[2 * wid + 1, 0, :])


_mesh = plsc.VectorSubcoreMesh(core_axis_name="c", subcore_axis_name="s",
                               num_cores=_NC, num_subcores=_NS)

_sc_call = pl.kernel(
    _sc_body,
    out_type=[
        jax.ShapeDtypeStruct((_B * _L, _D), jnp.float32),
        jax.ShapeDtypeStruct((_B * _L,), jnp.float32),
        jax.ShapeDtypeStruct((_B * _LP, _D), jnp.float32),
        jax.ShapeDtypeStruct((_B, 1, _LP), jnp.float32),
        jax.ShapeDtypeStruct((_VOCAB, _D), jnp.float32),
    ],
    mesh=_mesh,
    scratch_types=[
        pltpu.VMEM((2 * _L,), jnp.int32),
        pltpu.VMEM((2 * _L,), jnp.int32),
        pltpu.VMEM((_CH, _D), jnp.float32),
        pltpu.VMEM((_CH, _D), jnp.float32),
        pltpu.VMEM((_CH, _D), jnp.float32),
        pltpu.VMEM((_CH, _D), jnp.float32),
        pltpu.VMEM((_CH, _D), jnp.float32),
        pltpu.VMEM((_CH, _D), jnp.float32),
        pltpu.VMEM((2 * _L,), jnp.float32),
        pltpu.VMEM((_L,), jnp.float32),
        pltpu.VMEM((_L,), jnp.float32),
        pltpu.VMEM((_CH,), jnp.int32),
        pltpu.VMEM((_CH,), jnp.int32),
        pltpu.VMEM((_CH,), jnp.int32),
        pltpu.VMEM((_CH,), jnp.int32),
        pltpu.SemaphoreType.DMA,
        pltpu.SemaphoreType.DMA,
        pltpu.SemaphoreType.DMA,
        pltpu.SemaphoreType.DMA,
        pltpu.SemaphoreType.DMA,
        pltpu.SemaphoreType.DMA,
        pltpu.SemaphoreType.DMA,
        pltpu.SemaphoreType.DMA,
        pltpu.SemaphoreType.DMA,
        pltpu.SemaphoreType.DMA,
    ],
)


def kernel(x, y, emb, pos_src, pos_tgt):
    x = x.astype(jnp.int32)
    y = y.astype(jnp.int32)
    xe_flat, msk_flat, ye_flat, mtgt, _unused_scaled = _sc_call(
        x.reshape(-1), y.reshape(-1), emb, pos_src, pos_tgt)
    xe = xe_flat.reshape(_B, _L, _D)
    msk = msk_flat.reshape(_B, _L)
    labels = y[:, 1:]
    ye = ye_flat.reshape(_LP, _B, _D).transpose(1, 0, 2)
    return (xe, msk, msk, ye, mtgt.reshape(_B, _LP), labels)


# speedup vs baseline: 1.4870x; 1.4870x over previous
import math

import jax
import jax.numpy as jnp
from jax import lax
from jax.experimental import pallas as pl
from jax.experimental.pallas import tpu as pltpu
from jax.experimental.pallas import tpu_sc as plsc

_VOCAB = 1000
_D = 512
_B = 64
_L = 512
_SCALE = math.sqrt(float(_D))
_NEG_INF = float("-inf")

_NC = 2
_NS = 16
_NW = _NC * _NS
_CH = 32
_NP = _L // _CH
_LP = _L - 1
_NCK = 2 * _NP
_NT = _NCK // 4


def _mask16(iv):
    return jnp.where(iv == 0, jnp.float32(_NEG_INF), jnp.float32(0.0))


def _scale_add(rows_v, pos_v):
    def row_body(r, carry):
        for d in range(_D // 16):
            sl = pl.ds(d * 16, 16)
            rows_v[r, sl] = rows_v[r, sl] * _SCALE + pos_v[r, sl]
        return carry
    lax.fori_loop(0, _CH, row_body, 0, unroll=False)


def _sc_body(x_hbm, y_hbm, emb_hbm, psrc_hbm, ptgt_hbm,
             xe_hbm, msk_hbm, ye_hbm, mtgt_hbm,
             xidx_v, yidx_v,
             rows0, rows1, rows2, rows3, pos0, pos1,
             mskx_v, mska_v, mskb_v,
             sidx0, sidx1, sidx2, sidx3, pos_s,
             sg0, sg1, sg2, sg3, ss0, ss1, ss2, ss3, sp0, sp1):
    wid = lax.axis_index("s") * _NC + lax.axis_index("c")
    sid = lax.axis_index("s")
    iota16 = lax.iota(jnp.int32, 16)
    rows = [rows0, rows1, rows2, rows3]
    pos = [pos0, pos1]
    sidx = [sidx0, sidx1, sidx2, sidx3]
    sg = [sg0, sg1, sg2, sg3]
    ss = [ss0, ss1, ss2, ss3]
    sp = [sp0, sp1]
    base1k = wid * (2 * _L)

    def wait_gather(s):
        pltpu.make_async_copy(psrc_hbm.at[pl.ds(0, _CH), :], rows[s],
                              sg[s]).wait()

    def wait_store(s):
        pltpu.make_async_copy(rows[s], xe_hbm.at[pl.ds(0, _CH), :],
                              ss[s]).wait()

    def wait_pos(par):
        pltpu.make_async_copy(psrc_hbm.at[pl.ds(0, _CH), :], pos[par],
                              sp[par]).wait()

    pbase = pl.multiple_of(sid * _CH, _CH)
    pltpu.sync_copy(psrc_hbm.at[pl.ds(pbase, _CH), :], rows0)
    pltpu.sync_copy(rows0, pos_s.at[pl.ds(pbase, _CH), :])
    pltpu.sync_copy(x_hbm.at[pl.ds(base1k, 2 * _L)], xidx_v)
    pltpu.sync_copy(y_hbm.at[pl.ds(base1k, 2 * _L)], yidx_v)
    plsc.subcore_barrier()

    def run_loop(idx_v, pos_hbm, p_eff_fn, store_fn, mask_fn):
        def idx_off(p_lin, bi):
            l0 = p_eff_fn(p_lin) * _CH
            return pl.multiple_of(bi * _L + l0, _CH), l0

        def issue_gather(p_lin, bi, s):
            o_i, _ = idx_off(p_lin, bi)
            pltpu.async_copy(emb_hbm.at[idx_v.at[pl.ds(o_i, _CH)]],
                             rows[s], sg[s])

        def issue_pos(p_lin, par):
            l0 = pl.multiple_of(p_eff_fn(p_lin) * _CH, _CH)
            pltpu.async_copy(pos_hbm.at[pl.ds(l0, _CH), :],
                             pos[par], sp[par])

        issue_pos(0, 0)
        issue_gather(0, 0, 0)
        issue_gather(0, 1, 1)

        def group(t, carry):
            for s in range(4):
                s_half = s // 2
                bi = s % 2
                p_lin = 2 * t + s_half
                o_i, l0 = idx_off(p_lin, bi)
                if s == 0:
                    wait_pos(0)
                if s == 2:
                    wait_pos(1)
                wait_gather(s)
                _scale_add(rows[s], pos[s_half])
                store_fn(p_lin, bi, s, o_i, l0)
                mask_fn(bi, o_i, l0)
                if s == 0:
                    issue_pos(2 * t + 1, 1)
                if s == 2:
                    @pl.when(t < _NT - 1)
                    def _():
                        issue_pos(2 * t + 2, 0)
                j = (s + 2) % 4
                if s in (0, 1):
                    @pl.when(t > 0)
                    def _():
                        wait_store(j)
                    issue_gather(2 * t + 1, bi, j)
                else:
                    wait_store(j)

                    @pl.when(t < _NT - 1)
                    def _():
                        issue_gather(2 * t + 2, bi, j)
            return carry

        lax.fori_loop(0, _NT, group, 0, unroll=False)
        wait_store(2)
        wait_store(3)

    def xe_p_eff(p_lin):
        return p_lin

    def xe_store(p_lin, bi, s, o_i, l0):
        dst = pl.multiple_of(base1k + o_i, _CH)
        pltpu.async_copy(rows[s], xe_hbm.at[pl.ds(dst, _CH), :], ss[s])

    def xe_mask(bi, o_i, l0):
        for j in range(_CH // 16):
            mskx_v[pl.ds(o_i + j * 16, 16)] = _mask16(
                xidx_v[pl.ds(o_i + j * 16, 16)])

    run_loop(xidx_v, pos_s, xe_p_eff, xe_store, xe_mask)
    pltpu.sync_copy(mskx_v, msk_hbm.at[pl.ds(base1k, 2 * _L)])

    plsc.subcore_barrier()
    pltpu.sync_copy(ptgt_hbm.at[pl.ds(pbase, _CH), :], rows0)
    pltpu.sync_copy(rows0, pos_s.at[pl.ds(pbase, _CH), :])
    plsc.subcore_barrier()

    def ye_p_eff(p_lin):
        return jnp.where(
            p_lin == 0, _NP - 1,
            jnp.where(p_lin == 1, 1, jnp.where(p_lin == 2, 0, p_lin - 1)))

    def ye_store(p_lin, bi, s, o_i, l0):
        b = 2 * wid + bi
        for j in range(_CH // 16):
            posv = l0 + j * 16 + iota16
            sidx[s][j * 16:(j + 1) * 16] = jnp.where(
                posv == _L - 1, b, posv * _B + b)
        pltpu.async_copy(rows[s], ye_hbm.at[sidx[s]], ss[s])

    def ye_mask(bi, o_i, l0):
        mv = mska_v if bi == 0 else mskb_v
        for j in range(_CH // 16):
            mv[pl.ds(l0 + j * 16, 16)] = _mask16(
                yidx_v[pl.ds(o_i + j * 16, 16)])

    run_loop(yidx_v, pos_s, ye_p_eff, ye_store, ye_mask)
    pltpu.sync_copy(mska_v.at[pl.ds(0, _LP)], mtgt_hbm.at[2 * wid, 0, :])
    pltpu.sync_copy(mskb_v.at[pl.ds(0, _LP)],
                    mtgt_hbm.at[2 * wid + 1, 0, :])


_mesh = plsc.VectorSubcoreMesh(core_axis_name="c", subcore_axis_name="s",
                               num_cores=_NC, num_subcores=_NS)

_sc_call = pl.kernel(
    _sc_body,
    out_type=[
        jax.ShapeDtypeStruct((_B * _L, _D), jnp.float32),
        jax.ShapeDtypeStruct((_B * _L,), jnp.float32),
        jax.ShapeDtypeStruct((_B * _LP, _D), jnp.float32),
        jax.ShapeDtypeStruct((_B, 1, _LP), jnp.float32),
    ],
    mesh=_mesh,
    scratch_types=[
        pltpu.VMEM((2 * _L,), jnp.int32),
        pltpu.VMEM((2 * _L,), jnp.int32),
        pltpu.VMEM((_CH, _D), jnp.float32),
        pltpu.VMEM((_CH, _D), jnp.float32),
        pltpu.VMEM((_CH, _D), jnp.float32),
        pltpu.VMEM((_CH, _D), jnp.float32),
        pltpu.VMEM((_CH, _D), jnp.float32),
        pltpu.VMEM((_CH, _D), jnp.float32),
        pltpu.VMEM((2 * _L,), jnp.float32),
        pltpu.VMEM((_L,), jnp.float32),
        pltpu.VMEM((_L,), jnp.float32),
        pltpu.VMEM((_CH,), jnp.int32),
        pltpu.VMEM((_CH,), jnp.int32),
        pltpu.VMEM((_CH,), jnp.int32),
        pltpu.VMEM((_CH,), jnp.int32),
        pltpu.VMEM_SHARED((_L, _D), jnp.float32),
        pltpu.SemaphoreType.DMA,
        pltpu.SemaphoreType.DMA,
        pltpu.SemaphoreType.DMA,
        pltpu.SemaphoreType.DMA,
        pltpu.SemaphoreType.DMA,
        pltpu.SemaphoreType.DMA,
        pltpu.SemaphoreType.DMA,
        pltpu.SemaphoreType.DMA,
        pltpu.SemaphoreType.DMA,
        pltpu.SemaphoreType.DMA,
    ],
)


def kernel(x, y, emb, pos_src, pos_tgt):
    x = x.astype(jnp.int32)
    y = y.astype(jnp.int32)
    xe_flat, msk_flat, ye_flat, mtgt = _sc_call(
        x.reshape(-1), y.reshape(-1), emb, pos_src, pos_tgt)
    xe = xe_flat.reshape(_B, _L, _D)
    msk = msk_flat.reshape(_B, _L)
    labels = y[:, 1:]
    ye = ye_flat.reshape(_LP, _B, _D).transpose(1, 0, 2)
    return (xe, msk, msk, ye, mtgt.reshape(_B, _LP), labels)
